# baseline (speedup 1.0000x reference)
import jax
import jax.numpy as jnp
from jax import lax
from jax.experimental import pallas as pl
from jax.experimental.pallas import tpu as pltpu
from jax.experimental.pallas import tpu_sc as plsc

N = 10000
F = 128
E = 320000

NTILE = 16
K = 128
EPT = 20480
E_PAD = EPT * NTILE
ACC_ROWS = 10240
ZROWS = ACC_ROWS // NTILE
WPT = 624

R_BLK = 2000


K2 = 128
CH = EPT // K2
G = 16
SG = CH // G
NBUF = 2


def _seg_kernel(table, src5, dst5, zeros, out,
                srcg, dstg, r0, r1,
                acc, g0, g1, semi_s, semi_d):
    c = lax.axis_index("c")
    s = lax.axis_index("s")
    rows = (r0, r1)
    gsem = (g0, g1)
    pltpu.sync_copy(zeros, acc.at[pl.ds(s * ZROWS, ZROWS)])
    pltpu.sync_copy(src5.at[c, s, 0], srcg.at[0])
    pltpu.sync_copy(dst5.at[c, s, 0], dstg.at[0])
    plsc.subcore_barrier()

    for j in range(NBUF):
        pltpu.async_copy(table.at[srcg.at[0, j]], rows[j], gsem[j])

    def super_body(g, carry):
        p = lax.rem(g, 2)

        @pl.when(g + 1 < SG)
        def _():
            pltpu.async_copy(src5.at[c, s, g + 1], srcg.at[1 - p], semi_s)
            pltpu.async_copy(dst5.at[c, s, g + 1], dstg.at[1 - p], semi_d)

        for j in range(G):
            b = j % NBUF
            pltpu.make_async_copy(table.at[srcg.at[p, j]],
                                  rows[b], gsem[b]).wait()
            pltpu.sync_copy(rows[b], acc.at[dstg.at[p, j]], add=True)
            if j + NBUF < G:
                pltpu.async_copy(table.at[srcg.at[p, j + NBUF]],
                                 rows[b], gsem[b])
            else:
                nj = j + NBUF - G

                @pl.when(g + 1 < SG)
                def _(nj=nj, b=b):
                    if nj == 0:
                        pltpu.make_async_copy(src5.at[c, s, g + 1],
                                              srcg.at[1 - p], semi_s).wait()
                        pltpu.make_async_copy(dst5.at[c, s, g + 1],
                                              dstg.at[1 - p], semi_d).wait()
                    pltpu.async_copy(table.at[srcg.at[1 - p, nj]],
                                     rows[b], gsem[b])
        return carry

    lax.fori_loop(0, SG, super_body, 0)
    plsc.subcore_barrier()

    @pl.when(s < NTILE - 1)
    def _():
        pltpu.sync_copy(acc.at[pl.ds(s * WPT, WPT)],
                        out.at[pl.ds(c * N + s * WPT, WPT)])

    @pl.when(s == NTILE - 1)
    def _():
        last = (NTILE - 1) * WPT
        pltpu.sync_copy(acc.at[pl.ds(last, N - last)],
                        out.at[pl.ds(c * N + last, N - last)])


def _seg_sum2(table, srcall, dstall, zeros):
    mesh = plsc.VectorSubcoreMesh(core_axis_name="c", subcore_axis_name="s")
    f = pl.kernel(
        _seg_kernel,
        out_type=jax.ShapeDtypeStruct((2 * N, F), jnp.float32),
        mesh=mesh,
        scratch_types=[
            pltpu.VMEM((2, G, K2), jnp.int32),
            pltpu.VMEM((2, G, K2), jnp.int32),
            pltpu.VMEM((K2, F), jnp.float32),
            pltpu.VMEM((K2, F), jnp.float32),
            pltpu.VMEM_SHARED((ACC_ROWS, F), jnp.float32),
            pltpu.SemaphoreType.DMA,
            pltpu.SemaphoreType.DMA,
            pltpu.SemaphoreType.DMA,
            pltpu.SemaphoreType.DMA,
        ],
    )
    return f(table, srcall, dstall, zeros)


def _conv1_body(x_ref, agg_ref, w1, b1, w2, b2, out_ref):
    z = x_ref[...] + agg_ref[...]
    a = jnp.maximum(jnp.dot(z, w1[...], preferred_element_type=jnp.float32) + b1[...], 0.0)
    h = jnp.maximum(jnp.dot(a, w2[...], preferred_element_type=jnp.float32) + b2[...], 0.0)
    out_ref[...] = h


def _head_body(hs, hf, ags, agf, w3, b3, w4, b4, wa, ba, wa2t, wm, bm,
               wd, bd, gscale, beta, wpi, bpi, wvar, bvar, wmean, bmean,
               e1o, e2o, embo, pio, varo, meano):
    def tail(z):
        a = jnp.maximum(jnp.dot(z, w3[...], preferred_element_type=jnp.float32) + b3[...], 0.0)
        return jnp.dot(a, w4[...], preferred_element_type=jnp.float32) + b4[...]

    e1 = tail(hs[...] + ags[...])
    e2 = tail(hf[...] + agf[...])
    com = (e1 + e2) * 0.5

    def score(e):
        t = jnp.tanh(jnp.dot(e, wa[...], preferred_element_type=jnp.float32) + ba[...])
        return jnp.sum(t * wa2t[...], axis=1, keepdims=True)

    s1 = score(e1)
    s2 = score(com)
    s3 = score(e2)
    m = jnp.maximum(jnp.maximum(s1, s2), s3)
    x1 = jnp.exp(s1 - m)
    x2 = jnp.exp(s2 - m)
    x3 = jnp.exp(s3 - m)
    emb = (x1 * e1 + x2 * com + x3 * e2) / (x1 + x2 + x3)
    emb = jnp.dot(emb, wm[...], preferred_element_type=jnp.float32) + bm[...]
    e1o[...] = e1
    e2o[...] = e2
    embo[...] = emb
    hd = jnp.dot(emb, wd[...], preferred_element_type=jnp.float32) + bd[...]
    hd = jnp.maximum(hd * gscale[...] + beta[...], 0.0)
    pio[...] = jax.nn.sigmoid(jnp.dot(hd, wpi[...], preferred_element_type=jnp.float32) + bpi[...])
    varo[...] = jnp.clip(jax.nn.softplus(jnp.dot(hd, wvar[...], preferred_element_type=jnp.float32) + bvar[...]), 1e-4, 1e4)
    meano[...] = jnp.clip(jnp.exp(jnp.dot(hd, wmean[...], preferred_element_type=jnp.float32) + bmean[...]), 1e-5, 1e6)


def _pad_idx(a, fill):
    return jnp.concatenate([a, fill])


def kernel(x, sadj, fadj, W1, b1, W2, b2, W3, b3, W4, b4, Wa, ba, Wa2, Wm, bm,
           Wd, bd, gamma, beta, Wpi, bpi, Wvar, bvar, Wmean, bmean):
    f32 = jnp.float32
    NB = N // R_BLK

    idx4 = (2, NTILE, SG, G, K2)
    npad = E_PAD - E
    pad_src = (jnp.arange(npad, dtype=jnp.int32) * 13) % N
    pad_dst = N + (jnp.arange(npad, dtype=jnp.int32) % (ACC_ROWS - N))
    src1 = jnp.concatenate([_pad_idx(sadj[0], pad_src), _pad_idx(fadj[0], pad_src)]).reshape(idx4)
    src2 = jnp.concatenate([_pad_idx(sadj[0], pad_src), _pad_idx(fadj[0], pad_src) + N]).reshape(idx4)
    dsta = jnp.concatenate([_pad_idx(sadj[1], pad_dst), _pad_idx(fadj[1], pad_dst)]).reshape(idx4)
    zeros = jnp.zeros((ZROWS, F), f32)

    b1r = b1.reshape(1, F)
    b2r = b2.reshape(1, F)
    b3r = b3.reshape(1, F)
    b4r = b4.reshape(1, F)
    bar = ba.reshape(1, -1)
    wa2t = Wa2.reshape(1, -1)
    bmr = bm.reshape(1, F)
    bdr = bd.reshape(1, F)
    gscale = (gamma / jnp.sqrt(1.0 + 1e-5)).reshape(1, F)
    betar = beta.reshape(1, F)
    bpir = bpi.reshape(1, F)
    bvarr = bvar.reshape(1, F)
    bmeanr = bmean.reshape(1, F)

    agg1 = _seg_sum2(x, src1, dsta, zeros)

    wspec = pl.BlockSpec((F, F), lambda g, b: (0, 0))
    bspec = pl.BlockSpec((1, F), lambda g, b: (0, 0))
    h2 = pl.pallas_call(
        _conv1_body,
        grid=(2, NB),
        in_specs=[
            pl.BlockSpec((R_BLK, F), lambda g, b: (b, 0)),
            pl.BlockSpec((R_BLK, F), lambda g, b: (g * NB + b, 0)),
            wspec, bspec, wspec, bspec,
        ],
        out_specs=pl.BlockSpec((R_BLK, F), lambda g, b: (g * NB + b, 0)),
        out_shape=jax.ShapeDtypeStruct((2 * N, F), f32),
    )(x, agg1, W1, b1r, W2, b2r)

    agg2 = _seg_sum2(h2, src2, dsta, zeros)

    rs = pl.BlockSpec((R_BLK, F), lambda b: (b, 0))
    rf = pl.BlockSpec((R_BLK, F), lambda b: (NB + b, 0))
    w_ = lambda shape: pl.BlockSpec(shape, lambda b: (0, 0))
    outs = pl.pallas_call(
        _head_body,
        grid=(NB,),
        in_specs=[
            rs, rf, rs, rf,
            w_((F, F)), w_((1, F)), w_((F, F)), w_((1, F)),
            w_(Wa.shape), w_((1, Wa.shape[1])), w_(wa2t.shape),
            w_((F, F)), w_((1, F)),
            w_((F, F)), w_((1, F)), w_((1, F)), w_((1, F)),
            w_((F, F)), w_((1, F)),
            w_((F, F)), w_((1, F)),
            w_((F, F)), w_((1, F)),
        ],
        out_specs=[pl.BlockSpec((R_BLK, F), lambda b: (b, 0))] * 6,
        out_shape=[jax.ShapeDtypeStruct((N, F), f32)] * 6,
    )(h2, h2, agg2, agg2, W3, b3r, W4, b4r, Wa, bar, wa2t, Wm, bmr,
      Wd, bdr, gscale, betar, Wpi, bpir, Wvar, bvarr, Wmean, bmeanr)

    emb1, emb2, emb, pi, var, mean = outs
    return (emb1, emb2, emb, pi, var, mean)


# speedup vs baseline: 1.0712x; 1.0712x over previous
import jax
import jax.numpy as jnp
from jax import lax
from jax.experimental import pallas as pl
from jax.experimental.pallas import tpu as pltpu
from jax.experimental.pallas import tpu_sc as plsc

N = 10000
F = 128
E = 320000

NTILE = 16
K = 128
EPT = 20480
E_PAD = EPT * NTILE
ACC_ROWS = 10240
ZROWS = ACC_ROWS // NTILE
WPT = 624

R_BLK = 2000


K2 = 32
CH = EPT // K2
G = 16
SG = CH // G
NBUF = 8


def _seg_kernel(table, src5, dst5, zeros, out,
                srcg, dstg, r0, r1, r2, r3, r4, r5, r6, r7,
                acc, g0, g1, g2, g3, g4, g5, g6, g7, semi_s, semi_d):
    c = lax.axis_index("c")
    s = lax.axis_index("s")
    rows = (r0, r1, r2, r3, r4, r5, r6, r7)
    gsem = (g0, g1, g2, g3, g4, g5, g6, g7)
    pltpu.sync_copy(zeros, acc.at[pl.ds(s * ZROWS, ZROWS)])
    pltpu.sync_copy(src5.at[c, s, 0], srcg.at[0])
    pltpu.sync_copy(dst5.at[c, s, 0], dstg.at[0])
    plsc.subcore_barrier()

    for j in range(NBUF):
        pltpu.async_copy(table.at[srcg.at[0, j]], rows[j], gsem[j])

    def super_body(g, carry):
        p = lax.rem(g, 2)

        @pl.when(g + 1 < SG)
        def _():
            pltpu.async_copy(src5.at[c, s, g + 1], srcg.at[1 - p], semi_s)
            pltpu.async_copy(dst5.at[c, s, g + 1], dstg.at[1 - p], semi_d)

        for j in range(G):
            b = j % NBUF
            pltpu.make_async_copy(table.at[srcg.at[p, j]],
                                  rows[b], gsem[b]).wait()
            pltpu.sync_copy(rows[b], acc.at[dstg.at[p, j]], add=True)
            if j + NBUF < G:
                pltpu.async_copy(table.at[srcg.at[p, j + NBUF]],
                                 rows[b], gsem[b])
            else:
                nj = j + NBUF - G

                @pl.when(g + 1 < SG)
                def _(nj=nj, b=b):
                    if nj == 0:
                        pltpu.make_async_copy(src5.at[c, s, g + 1],
                                              srcg.at[1 - p], semi_s).wait()
                        pltpu.make_async_copy(dst5.at[c, s, g + 1],
                                              dstg.at[1 - p], semi_d).wait()
                    pltpu.async_copy(table.at[srcg.at[1 - p, nj]],
                                     rows[b], gsem[b])
        return carry

    lax.fori_loop(0, SG, super_body, 0)
    plsc.subcore_barrier()

    @pl.when(s < NTILE - 1)
    def _():
        pltpu.sync_copy(acc.at[pl.ds(s * WPT, WPT)],
                        out.at[pl.ds(c * N + s * WPT, WPT)])

    @pl.when(s == NTILE - 1)
    def _():
        last = (NTILE - 1) * WPT
        pltpu.sync_copy(acc.at[pl.ds(last, N - last)],
                        out.at[pl.ds(c * N + last, N - last)])


def _seg_sum2(table, srcall, dstall, zeros):
    mesh = plsc.VectorSubcoreMesh(core_axis_name="c", subcore_axis_name="s")
    f = pl.kernel(
        _seg_kernel,
        out_type=jax.ShapeDtypeStruct((2 * N, F), jnp.float32),
        mesh=mesh,
        scratch_types=[
            pltpu.VMEM((2, G, K2), jnp.int32),
            pltpu.VMEM((2, G, K2), jnp.int32),
            pltpu.VMEM((K2, F), jnp.float32),
            pltpu.VMEM((K2, F), jnp.float32),
            pltpu.VMEM((K2, F), jnp.float32),
            pltpu.VMEM((K2, F), jnp.float32),
            pltpu.VMEM((K2, F), jnp.float32),
            pltpu.VMEM((K2, F), jnp.float32),
            pltpu.VMEM((K2, F), jnp.float32),
            pltpu.VMEM((K2, F), jnp.float32),
            pltpu.VMEM_SHARED((ACC_ROWS, F), jnp.float32),
            pltpu.SemaphoreType.DMA,
            pltpu.SemaphoreType.DMA,
            pltpu.SemaphoreType.DMA,
            pltpu.SemaphoreType.DMA,
            pltpu.SemaphoreType.DMA,
            pltpu.SemaphoreType.DMA,
            pltpu.SemaphoreType.DMA,
            pltpu.SemaphoreType.DMA,
            pltpu.SemaphoreType.DMA,
            pltpu.SemaphoreType.DMA,
        ],
    )
    return f(table, srcall, dstall, zeros)


def _conv1_body(x_ref, agg_ref, w1, b1, w2, b2, out_ref):
    z = x_ref[...] + agg_ref[...]
    a = jnp.maximum(jnp.dot(z, w1[...], preferred_element_type=jnp.float32) + b1[...], 0.0)
    h = jnp.maximum(jnp.dot(a, w2[...], preferred_element_type=jnp.float32) + b2[...], 0.0)
    out_ref[...] = h


def _head_body(hs, hf, ags, agf, w3, b3, w4, b4, wa, ba, wa2t, wm, bm,
               wd, bd, gscale, beta, wpi, bpi, wvar, bvar, wmean, bmean,
               e1o, e2o, embo, pio, varo, meano):
    def tail(z):
        a = jnp.maximum(jnp.dot(z, w3[...], preferred_element_type=jnp.float32) + b3[...], 0.0)
        return jnp.dot(a, w4[...], preferred_element_type=jnp.float32) + b4[...]

    e1 = tail(hs[...] + ags[...])
    e2 = tail(hf[...] + agf[...])
    com = (e1 + e2) * 0.5

    def score(e):
        t = jnp.tanh(jnp.dot(e, wa[...], preferred_element_type=jnp.float32) + ba[...])
        return jnp.sum(t * wa2t[...], axis=1, keepdims=True)

    s1 = score(e1)
    s2 = score(com)
    s3 = score(e2)
    m = jnp.maximum(jnp.maximum(s1, s2), s3)
    x1 = jnp.exp(s1 - m)
    x2 = jnp.exp(s2 - m)
    x3 = jnp.exp(s3 - m)
    emb = (x1 * e1 + x2 * com + x3 * e2) / (x1 + x2 + x3)
    emb = jnp.dot(emb, wm[...], preferred_element_type=jnp.float32) + bm[...]
    e1o[...] = e1
    e2o[...] = e2
    embo[...] = emb
    hd = jnp.dot(emb, wd[...], preferred_element_type=jnp.float32) + bd[...]
    hd = jnp.maximum(hd * gscale[...] + beta[...], 0.0)
    pio[...] = jax.nn.sigmoid(jnp.dot(hd, wpi[...], preferred_element_type=jnp.float32) + bpi[...])
    varo[...] = jnp.clip(jax.nn.softplus(jnp.dot(hd, wvar[...], preferred_element_type=jnp.float32) + bvar[...]), 1e-4, 1e4)
    meano[...] = jnp.clip(jnp.exp(jnp.dot(hd, wmean[...], preferred_element_type=jnp.float32) + bmean[...]), 1e-5, 1e6)


def _pad_idx(a, fill):
    return jnp.concatenate([a, fill])


def kernel(x, sadj, fadj, W1, b1, W2, b2, W3, b3, W4, b4, Wa, ba, Wa2, Wm, bm,
           Wd, bd, gamma, beta, Wpi, bpi, Wvar, bvar, Wmean, bmean):
    f32 = jnp.float32
    NB = N // R_BLK

    idx4 = (2, NTILE, SG, G, K2)
    npad = E_PAD - E
    pad_src = (jnp.arange(npad, dtype=jnp.int32) * 13) % N
    pad_dst = N + (jnp.arange(npad, dtype=jnp.int32) % (ACC_ROWS - N))
    src1 = jnp.concatenate([_pad_idx(sadj[0], pad_src), _pad_idx(fadj[0], pad_src)]).reshape(idx4)
    src2 = jnp.concatenate([_pad_idx(sadj[0], pad_src), _pad_idx(fadj[0], pad_src) + N]).reshape(idx4)
    dsta = jnp.concatenate([_pad_idx(sadj[1], pad_dst), _pad_idx(fadj[1], pad_dst)]).reshape(idx4)
    zeros = jnp.zeros((ZROWS, F), f32)

    b1r = b1.reshape(1, F)
    b2r = b2.reshape(1, F)
    b3r = b3.reshape(1, F)
    b4r = b4.reshape(1, F)
    bar = ba.reshape(1, -1)
    wa2t = Wa2.reshape(1, -1)
    bmr = bm.reshape(1, F)
    bdr = bd.reshape(1, F)
    gscale = (gamma / jnp.sqrt(1.0 + 1e-5)).reshape(1, F)
    betar = beta.reshape(1, F)
    bpir = bpi.reshape(1, F)
    bvarr = bvar.reshape(1, F)
    bmeanr = bmean.reshape(1, F)

    agg1 = _seg_sum2(x, src1, dsta, zeros)

    wspec = pl.BlockSpec((F, F), lambda g, b: (0, 0))
    bspec = pl.BlockSpec((1, F), lambda g, b: (0, 0))
    h2 = pl.pallas_call(
        _conv1_body,
        grid=(2, NB),
        in_specs=[
            pl.BlockSpec((R_BLK, F), lambda g, b: (b, 0)),
            pl.BlockSpec((R_BLK, F), lambda g, b: (g * NB + b, 0)),
            wspec, bspec, wspec, bspec,
        ],
        out_specs=pl.BlockSpec((R_BLK, F), lambda g, b: (g * NB + b, 0)),
        out_shape=jax.ShapeDtypeStruct((2 * N, F), f32),
    )(x, agg1, W1, b1r, W2, b2r)

    agg2 = _seg_sum2(h2, src2, dsta, zeros)

    rs = pl.BlockSpec((R_BLK, F), lambda b: (b, 0))
    rf = pl.BlockSpec((R_BLK, F), lambda b: (NB + b, 0))
    w_ = lambda shape: pl.BlockSpec(shape, lambda b: (0, 0))
    outs = pl.pallas_call(
        _head_body,
        grid=(NB,),
        in_specs=[
            rs, rf, rs, rf,
            w_((F, F)), w_((1, F)), w_((F, F)), w_((1, F)),
            w_(Wa.shape), w_((1, Wa.shape[1])), w_(wa2t.shape),
            w_((F, F)), w_((1, F)),
            w_((F, F)), w_((1, F)), w_((1, F)), w_((1, F)),
            w_((F, F)), w_((1, F)),
            w_((F, F)), w_((1, F)),
            w_((F, F)), w_((1, F)),
        ],
        out_specs=[pl.BlockSpec((R_BLK, F), lambda b: (b, 0))] * 6,
        out_shape=[jax.ShapeDtypeStruct((N, F), f32)] * 6,
    )(h2, h2, agg2, agg2, W3, b3r, W4, b4r, Wa, bar, wa2t, Wm, bmr,
      Wd, bdr, gscale, betar, Wpi, bpir, Wvar, bvarr, Wmean, bmeanr)

    emb1, emb2, emb, pi, var, mean = outs
    return (emb1, emb2, emb, pi, var, mean)


# speedup vs baseline: 1.1005x; 1.0274x over previous
import jax
import jax.numpy as jnp
from jax import lax
from jax.experimental import pallas as pl
from jax.experimental.pallas import tpu as pltpu
from jax.experimental.pallas import tpu_sc as plsc

N = 10000
F = 128
E = 320000

NTILE = 16
K = 128
EPT = 20480
E_PAD = EPT * NTILE
ACC_ROWS = 10240
ZROWS = ACC_ROWS // NTILE
WPT = 624

R_BLK = 2000


K2 = 64
CH = EPT // K2
G = 16
SG = CH // G
NBUF = 4


def _seg_kernel(table, src5, dst5, zeros, out,
                srcg, dstg, r0, r1, r2, r3,
                acc, g0, g1, g2, g3, semi_s, semi_d):
    c = lax.axis_index("c")
    s = lax.axis_index("s")
    rows = (r0, r1, r2, r3)
    gsem = (g0, g1, g2, g3)
    pltpu.sync_copy(zeros, acc.at[pl.ds(s * ZROWS, ZROWS)])
    pltpu.sync_copy(src5.at[c, s, 0], srcg.at[0])
    pltpu.sync_copy(dst5.at[c, s, 0], dstg.at[0])
    plsc.subcore_barrier()

    for j in range(NBUF):
        pltpu.async_copy(table.at[srcg.at[0, j]], rows[j], gsem[j])

    def super_body(g, carry):
        p = lax.rem(g, 2)

        @pl.when(g + 1 < SG)
        def _():
            pltpu.async_copy(src5.at[c, s, g + 1], srcg.at[1 - p], semi_s)
            pltpu.async_copy(dst5.at[c, s, g + 1], dstg.at[1 - p], semi_d)

        for j in range(G):
            b = j % NBUF
            pltpu.make_async_copy(table.at[srcg.at[p, j]],
                                  rows[b], gsem[b]).wait()
            pltpu.sync_copy(rows[b], acc.at[dstg.at[p, j]], add=True)
            if j + NBUF < G:
                pltpu.async_copy(table.at[srcg.at[p, j + NBUF]],
                                 rows[b], gsem[b])
            else:
                nj = j + NBUF - G

                @pl.when(g + 1 < SG)
                def _(nj=nj, b=b):
                    if nj == 0:
                        pltpu.make_async_copy(src5.at[c, s, g + 1],
                                              srcg.at[1 - p], semi_s).wait()
                        pltpu.make_async_copy(dst5.at[c, s, g + 1],
                                              dstg.at[1 - p], semi_d).wait()
                    pltpu.async_copy(table.at[srcg.at[1 - p, nj]],
                                     rows[b], gsem[b])
        return carry

    lax.fori_loop(0, SG, super_body, 0)
    plsc.subcore_barrier()

    @pl.when(s < NTILE - 1)
    def _():
        pltpu.sync_copy(acc.at[pl.ds(s * WPT, WPT)],
                        out.at[pl.ds(c * N + s * WPT, WPT)])

    @pl.when(s == NTILE - 1)
    def _():
        last = (NTILE - 1) * WPT
        pltpu.sync_copy(acc.at[pl.ds(last, N - last)],
                        out.at[pl.ds(c * N + last, N - last)])


def _seg_sum2(table, srcall, dstall, zeros):
    mesh = plsc.VectorSubcoreMesh(core_axis_name="c", subcore_axis_name="s")
    f = pl.kernel(
        _seg_kernel,
        out_type=jax.ShapeDtypeStruct((2 * N, F), jnp.float32),
        mesh=mesh,
        scratch_types=[
            pltpu.VMEM((2, G, K2), jnp.int32),
            pltpu.VMEM((2, G, K2), jnp.int32),
            pltpu.VMEM((K2, F), jnp.float32),
            pltpu.VMEM((K2, F), jnp.float32),
            pltpu.VMEM((K2, F), jnp.float32),
            pltpu.VMEM((K2, F), jnp.float32),
            pltpu.VMEM_SHARED((ACC_ROWS, F), jnp.float32),
            pltpu.SemaphoreType.DMA,
            pltpu.SemaphoreType.DMA,
            pltpu.SemaphoreType.DMA,
            pltpu.SemaphoreType.DMA,
            pltpu.SemaphoreType.DMA,
            pltpu.SemaphoreType.DMA,
        ],
    )
    return f(table, srcall, dstall, zeros)


def _conv1_body(x_ref, agg_ref, w1, b1, w2, b2, out_ref):
    z = x_ref[...] + agg_ref[...]
    a = jnp.maximum(jnp.dot(z, w1[...], preferred_element_type=jnp.float32) + b1[...], 0.0)
    h = jnp.maximum(jnp.dot(a, w2[...], preferred_element_type=jnp.float32) + b2[...], 0.0)
    out_ref[...] = h


def _head_body(hs, hf, ags, agf, w3, b3, w4, b4, wa, ba, wa2t, wm, bm,
               wd, bd, gscale, beta, wpi, bpi, wvar, bvar, wmean, bmean,
               e1o, e2o, embo, pio, varo, meano):
    def tail(z):
        a = jnp.maximum(jnp.dot(z, w3[...], preferred_element_type=jnp.float32) + b3[...], 0.0)
        return jnp.dot(a, w4[...], preferred_element_type=jnp.float32) + b4[...]

    e1 = tail(hs[...] + ags[...])
    e2 = tail(hf[...] + agf[...])
    com = (e1 + e2) * 0.5

    def score(e):
        t = jnp.tanh(jnp.dot(e, wa[...], preferred_element_type=jnp.float32) + ba[...])
        return jnp.sum(t * wa2t[...], axis=1, keepdims=True)

    s1 = score(e1)
    s2 = score(com)
    s3 = score(e2)
    m = jnp.maximum(jnp.maximum(s1, s2), s3)
    x1 = jnp.exp(s1 - m)
    x2 = jnp.exp(s2 - m)
    x3 = jnp.exp(s3 - m)
    emb = (x1 * e1 + x2 * com + x3 * e2) / (x1 + x2 + x3)
    emb = jnp.dot(emb, wm[...], preferred_element_type=jnp.float32) + bm[...]
    e1o[...] = e1
    e2o[...] = e2
    embo[...] = emb
    hd = jnp.dot(emb, wd[...], preferred_element_type=jnp.float32) + bd[...]
    hd = jnp.maximum(hd * gscale[...] + beta[...], 0.0)
    pio[...] = jax.nn.sigmoid(jnp.dot(hd, wpi[...], preferred_element_type=jnp.float32) + bpi[...])
    varo[...] = jnp.clip(jax.nn.softplus(jnp.dot(hd, wvar[...], preferred_element_type=jnp.float32) + bvar[...]), 1e-4, 1e4)
    meano[...] = jnp.clip(jnp.exp(jnp.dot(hd, wmean[...], preferred_element_type=jnp.float32) + bmean[...]), 1e-5, 1e6)


def _pad_idx(a, fill):
    return jnp.concatenate([a, fill])


def kernel(x, sadj, fadj, W1, b1, W2, b2, W3, b3, W4, b4, Wa, ba, Wa2, Wm, bm,
           Wd, bd, gamma, beta, Wpi, bpi, Wvar, bvar, Wmean, bmean):
    f32 = jnp.float32
    NB = N // R_BLK

    idx4 = (2, NTILE, SG, G, K2)
    npad = E_PAD - E
    pad_src = (jnp.arange(npad, dtype=jnp.int32) * 13) % N
    pad_dst = N + (jnp.arange(npad, dtype=jnp.int32) % (ACC_ROWS - N))
    src1 = jnp.concatenate([_pad_idx(sadj[0], pad_src), _pad_idx(fadj[0], pad_src)]).reshape(idx4)
    src2 = jnp.concatenate([_pad_idx(sadj[0], pad_src), _pad_idx(fadj[0], pad_src) + N]).reshape(idx4)
    dsta = jnp.concatenate([_pad_idx(sadj[1], pad_dst), _pad_idx(fadj[1], pad_dst)]).reshape(idx4)
    zeros = jnp.zeros((ZROWS, F), f32)

    b1r = b1.reshape(1, F)
    b2r = b2.reshape(1, F)
    b3r = b3.reshape(1, F)
    b4r = b4.reshape(1, F)
    bar = ba.reshape(1, -1)
    wa2t = Wa2.reshape(1, -1)
    bmr = bm.reshape(1, F)
    bdr = bd.reshape(1, F)
    gscale = (gamma / jnp.sqrt(1.0 + 1e-5)).reshape(1, F)
    betar = beta.reshape(1, F)
    bpir = bpi.reshape(1, F)
    bvarr = bvar.reshape(1, F)
    bmeanr = bmean.reshape(1, F)

    agg1 = _seg_sum2(x, src1, dsta, zeros)

    wspec = pl.BlockSpec((F, F), lambda g, b: (0, 0))
    bspec = pl.BlockSpec((1, F), lambda g, b: (0, 0))
    h2 = pl.pallas_call(
        _conv1_body,
        grid=(2, NB),
        in_specs=[
            pl.BlockSpec((R_BLK, F), lambda g, b: (b, 0)),
            pl.BlockSpec((R_BLK, F), lambda g, b: (g * NB + b, 0)),
            wspec, bspec, wspec, bspec,
        ],
        out_specs=pl.BlockSpec((R_BLK, F), lambda g, b: (g * NB + b, 0)),
        out_shape=jax.ShapeDtypeStruct((2 * N, F), f32),
    )(x, agg1, W1, b1r, W2, b2r)

    agg2 = _seg_sum2(h2, src2, dsta, zeros)

    rs = pl.BlockSpec((R_BLK, F), lambda b: (b, 0))
    rf = pl.BlockSpec((R_BLK, F), lambda b: (NB + b, 0))
    w_ = lambda shape: pl.BlockSpec(shape, lambda b: (0, 0))
    outs = pl.pallas_call(
        _head_body,
        grid=(NB,),
        in_specs=[
            rs, rf, rs, rf,
            w_((F, F)), w_((1, F)), w_((F, F)), w_((1, F)),
            w_(Wa.shape), w_((1, Wa.shape[1])), w_(wa2t.shape),
            w_((F, F)), w_((1, F)),
            w_((F, F)), w_((1, F)), w_((1, F)), w_((1, F)),
            w_((F, F)), w_((1, F)),
            w_((F, F)), w_((1, F)),
            w_((F, F)), w_((1, F)),
        ],
        out_specs=[pl.BlockSpec((R_BLK, F), lambda b: (b, 0))] * 6,
        out_shape=[jax.ShapeDtypeStruct((N, F), f32)] * 6,
    )(h2, h2, agg2, agg2, W3, b3r, W4, b4r, Wa, bar, wa2t, Wm, bmr,
      Wd, bdr, gscale, betar, Wpi, bpir, Wvar, bvarr, Wmean, bmeanr)

    emb1, emb2, emb, pi, var, mean = outs
    return (emb1, emb2, emb, pi, var, mean)
